# baseline (speedup 1.0000x reference)
import functools

import jax
import jax.numpy as jnp
from jax.experimental import pallas as pl
from jax.experimental.pallas import tpu as pltpu

B, N, D_IN, D_E, OUT, TGT = 64, 32, 128, 16, 128, 12
NDEG = 33
MSG = D_IN + D_E
EW = N * D_E
GPC = 8
CH = GPC * N
PACK = 4
NG = (NDEG + PACK - 1) // PACK

_F32 = jnp.float32
_BF16 = jnp.bfloat16


def _dot(a, b):
    return jax.lax.dot_general(
        a, b, (((1,), (0,)), ((), ())), preferred_element_type=_F32)


def _dot_bf(a, b):
    return _dot(a.astype(_BF16), b.astype(_BF16))


def _gnn_kernel(g_ref, e_ref, h_ref, Hb_ref,
                W0_ref, W1_ref, W2_ref, W3_ref,
                nW0_ref, nb0_ref, nW1_ref, nb1_ref,
                nW2_ref, nb2_ref, nW3_ref, nb3_ref,
                out_ref, acc_ref):
    g = g_ref[...]
    deg = jnp.sum(g, axis=1, keepdims=True)
    deg = jnp.minimum(deg, float(NDEG - 1))

    tq = jax.lax.broadcasted_iota(jnp.int32, (N, CH), 1)
    tw = jax.lax.broadcasted_iota(jnp.int32, (N, CH), 0)
    T = (tq % N == tw).astype(_BF16)
    ri = jax.lax.broadcasted_iota(jnp.int32, (CH, CH), 0)
    ci = jax.lax.broadcasted_iota(jnp.int32, (CH, CH), 1)
    blk = (ri // N == ci // N).astype(_F32)
    gbd = (_dot(g.astype(_BF16), T) * blk).astype(_BF16)

    rl = jax.lax.broadcasted_iota(jnp.int32, (N, EW), 1)
    rw = jax.lax.broadcasted_iota(jnp.int32, (N, EW), 0)
    R = (rl // D_E == rw).astype(_BF16)
    sl = jax.lax.broadcasted_iota(jnp.int32, (EW, D_E), 0)
    sj = jax.lax.broadcasted_iota(jnp.int32, (EW, D_E), 1)
    S = (sl % D_E == sj).astype(_BF16)
    g_rep = _dot(g.astype(_BF16), R)
    prod = g_rep * e_ref[...]
    p_hi = prod.astype(_BF16)
    p_lo = (prod - p_hi.astype(_F32)).astype(_BF16)
    m_e = _dot(p_hi, S) + _dot(p_lo, S)

    si = jax.lax.broadcasted_iota(jnp.int32, (GPC, CH), 0)
    sp = jax.lax.broadcasted_iota(jnp.int32, (GPC, CH), 1)
    Ssum = (sp // N == si).astype(_F32)

    def readout(h_l, W_ref):
        mask = (jnp.sum(h_l, axis=1, keepdims=True) != 0).astype(_F32)
        hsum = _dot(Ssum, h_l * mask)
        return _dot(hsum, W_ref[...])

    grp = jnp.floor(deg * (1.0 / PACK))

    h = h_ref[...]
    aux = readout(h, W0_ref)

    for layer, W_ref in ((0, W1_ref), (1, W2_ref), (2, W3_ref)):
        m_h = _dot(gbd, h.astype(_BF16))
        m = jnp.concatenate([m_h, m_e], axis=1)
        m_bf = m.astype(_BF16)
        acc_ref[...] = jnp.zeros((CH, OUT), dtype=_F32)
        for gi in range(NG):
            lo_d = gi * PACK
            n_d = min(PACK, NDEG - lo_d)
            present = jnp.max(jnp.where(grp == float(gi), 1.0, 0.0))

            @pl.when(present > 0.0)
            def _():
                parts = [m_bf * (deg == float(lo_d + k)).astype(_BF16)
                         for k in range(n_d)]
                m4 = jnp.concatenate(parts, axis=1)
                Hp = Hb_ref[layer, lo_d * MSG:(lo_d + n_d) * MSG, :]
                acc_ref[...] += _dot(m4, Hp)
        h = jax.nn.sigmoid(acc_ref[...])
        aux = aux + readout(h, W_ref)

    s = jax.nn.softmax(aux, axis=1)
    x = jax.nn.relu(_dot(s, nW0_ref[...]) + nb0_ref[...])
    x = jax.nn.relu(_dot(x, nW1_ref[...]) + nb1_ref[...])
    x = jax.nn.relu(_dot(x, nW2_ref[...]) + nb2_ref[...])
    x = jax.nn.sigmoid(_dot(x, nW3_ref[...]) + nb3_ref[...])
    out_ref[...] = jax.nn.softmax(x, axis=1)


@functools.partial(jax.jit, static_argnames=("interpret",))
def _run(g, h_in, e, H0, H1, H2, W0, W1, W2, W3,
         nW0, nb0, nW1, nb1, nW2, nb2, nW3, nb3, interpret=False):
    g2 = g.reshape(B * N, N)
    e2 = e.reshape(B * N, EW)
    h2 = h_in.reshape(B * N, D_IN)
    Hb = jnp.stack([H0.reshape(NDEG * MSG, OUT),
                    H1.reshape(NDEG * MSG, OUT),
                    H2.reshape(NDEG * MSG, OUT)]).astype(_BF16)
    const = lambda shape: pl.BlockSpec(shape, lambda i: (0,) * len(shape))
    return pl.pallas_call(
        _gnn_kernel,
        grid=(B // GPC,),
        in_specs=[
            pl.BlockSpec((CH, N), lambda i: (i, 0)),
            pl.BlockSpec((CH, EW), lambda i: (i, 0)),
            pl.BlockSpec((CH, D_IN), lambda i: (i, 0)),
            const((3, NDEG * MSG, OUT)),
            const((D_IN, OUT)), const((OUT, OUT)),
            const((OUT, OUT)), const((OUT, OUT)),
            const((OUT, 128)), const((1, 128)),
            const((128, 256)), const((1, 256)),
            const((256, 128)), const((1, 128)),
            const((128, TGT)), const((1, TGT)),
        ],
        out_specs=pl.BlockSpec((GPC, TGT), lambda i: (i, 0)),
        out_shape=jax.ShapeDtypeStruct((B, TGT), _F32),
        scratch_shapes=[pltpu.VMEM((CH, OUT), _F32)],
        interpret=interpret,
    )(g2, e2, h2, Hb, W0, W1, W2, W3,
      nW0, nb0.reshape(1, -1), nW1, nb1.reshape(1, -1),
      nW2, nb2.reshape(1, -1), nW3, nb3.reshape(1, -1))


def kernel(g, h_in, e, H0, H1, H2, W0, W1, W2, W3,
           nW0, nb0, nW1, nb1, nW2, nb2, nW3, nb3):
    return _run(g, h_in, e, H0, H1, H2, W0, W1, W2, W3,
                nW0, nb0, nW1, nb1, nW2, nb2, nW3, nb3)


# speedup vs baseline: 1.2390x; 1.2390x over previous
import functools

import jax
import jax.numpy as jnp
from jax.experimental import pallas as pl
from jax.experimental.pallas import tpu as pltpu

B, N, D_IN, D_E, OUT, TGT = 64, 32, 128, 16, 128, 12
NDEG = 33
P = B * N
MSG = D_IN + D_E
EW = N * D_E
CH = 256

_F32 = jnp.float32
_BF16 = jnp.bfloat16


def _dot(a, b):
    return jax.lax.dot_general(
        a, b, (((1,), (0,)), ((), ())), preferred_element_type=_F32)


def _dot_bf(a, b):
    return _dot(a.astype(_BF16), b.astype(_BF16))


def _gnn_kernel(g_ref, e_ref, h_ref, Ht_ref, Hb_ref,
                W0_ref, W1_ref, W2_ref, W3_ref,
                nW0_ref, nb0_ref, nW1_ref, nb1_ref,
                nW2_ref, nb2_ref, nW3_ref, nb3_ref,
                out_ref, gbd_ref):
    g = g_ref[...]
    deg = jnp.sum(g, axis=1, keepdims=True)
    deg = jnp.minimum(deg, float(NDEG - 1))

    tq = jax.lax.broadcasted_iota(jnp.int32, (N, CH), 1)
    tw = jax.lax.broadcasted_iota(jnp.int32, (N, CH), 0)
    T = (tq % N == tw).astype(_BF16)
    ri = jax.lax.broadcasted_iota(jnp.int32, (CH, CH), 0)
    ci = jax.lax.broadcasted_iota(jnp.int32, (CH, CH), 1)
    blk = (ri // N == ci // N).astype(_BF16)
    for c in range(P // CH):
        rows = _dot(g_ref[c * CH:(c + 1) * CH, :].astype(_BF16), T)
        gbd_ref[c * CH:(c + 1) * CH, :] = rows.astype(_BF16) * blk

    rl = jax.lax.broadcasted_iota(jnp.int32, (N, EW), 1)
    rw = jax.lax.broadcasted_iota(jnp.int32, (N, EW), 0)
    R = (rl // D_E == rw).astype(_BF16)
    sl = jax.lax.broadcasted_iota(jnp.int32, (EW, D_E), 0)
    sj = jax.lax.broadcasted_iota(jnp.int32, (EW, D_E), 1)
    S = (sl % D_E == sj).astype(_BF16)
    g_rep = _dot(g.astype(_BF16), R)
    prod = g_rep * e_ref[...]
    p_hi = prod.astype(_BF16)
    p_lo = (prod - p_hi.astype(_F32)).astype(_BF16)
    m_e = _dot(p_hi, S) + _dot(p_lo, S)
    m_e_bf = m_e.astype(_BF16)

    dmask = [(deg == float(d)).astype(_BF16) for d in range(NDEG)]

    def readout(h_l, W_ref):
        mask = (jnp.sum(h_l, axis=1, keepdims=True) != 0).astype(_F32)
        hm = h_l * mask
        hsum = jnp.sum(hm.reshape(B, N, OUT), axis=1)
        return _dot(hsum, W_ref[...])

    h = h_ref[...]
    aux = readout(h, W0_ref)

    for H_ref, W_ref in ((0, W1_ref), (1, W2_ref), (2, W3_ref)):
        h_bf = h.astype(_BF16)
        mh_cs = [_dot(gbd_ref[c * CH:(c + 1) * CH, :],
                      h_bf[c * CH:(c + 1) * CH, :]) for c in range(P // CH)]
        m_h = jnp.concatenate(mh_cs, axis=0)
        m_h_bf = m_h.astype(_BF16)
        lhs_h = jnp.concatenate([m_h_bf * dm for dm in dmask], axis=1)
        lhs_e = jnp.concatenate([m_e_bf * dm for dm in dmask], axis=1)
        acc = (_dot(lhs_h, Ht_ref[H_ref]) + _dot(lhs_e, Hb_ref[H_ref]))
        h = jax.nn.sigmoid(acc)
        aux = aux + readout(h, W_ref)

    s = jax.nn.softmax(aux, axis=1)
    x = jax.nn.relu(_dot(s, nW0_ref[...]) + nb0_ref[...])
    x = jax.nn.relu(_dot(x, nW1_ref[...]) + nb1_ref[...])
    x = jax.nn.relu(_dot(x, nW2_ref[...]) + nb2_ref[...])
    x = jax.nn.sigmoid(_dot(x, nW3_ref[...]) + nb3_ref[...])
    out_ref[...] = jax.nn.softmax(x, axis=1)


@functools.partial(jax.jit, static_argnames=("interpret",))
def _run(g, h_in, e, H0, H1, H2, W0, W1, W2, W3,
         nW0, nb0, nW1, nb1, nW2, nb2, nW3, nb3, interpret=False):
    g2 = g.reshape(P, N)
    e2 = e.reshape(P, EW)
    h2 = h_in.reshape(P, D_IN)
    Hs = jnp.stack([H0, H1, H2])
    Ht = Hs[:, :, :D_IN, :].reshape(3, NDEG * D_IN, OUT).astype(_BF16)
    Hb = Hs[:, :, D_IN:, :].reshape(3, NDEG * D_E, OUT).astype(_BF16)
    return pl.pallas_call(
        _gnn_kernel,
        out_shape=jax.ShapeDtypeStruct((B, TGT), _F32),
        scratch_shapes=[pltpu.VMEM((P, CH), _BF16)],
        interpret=interpret,
    )(g2, e2, h2, Ht, Hb, W0, W1, W2, W3,
      nW0, nb0.reshape(1, -1), nW1, nb1.reshape(1, -1),
      nW2, nb2.reshape(1, -1), nW3, nb3.reshape(1, -1))


def kernel(g, h_in, e, H0, H1, H2, W0, W1, W2, W3,
           nW0, nb0, nW1, nb1, nW2, nb2, nW3, nb3):
    return _run(g, h_in, e, H0, H1, H2, W0, W1, W2, W3,
                nW0, nb0, nW1, nb1, nW2, nb2, nW3, nb3)


# speedup vs baseline: 1.2686x; 1.0239x over previous
import functools

import jax
import jax.numpy as jnp
from jax.experimental import pallas as pl
from jax.experimental.pallas import tpu as pltpu

B, N, D_IN, D_E, OUT, TGT = 64, 32, 128, 16, 128, 12
NDEG = 33
P = B * N
MSG = D_IN + D_E
EW = N * D_E
CH = 256

_F32 = jnp.float32
_BF16 = jnp.bfloat16


def _dot(a, b):
    return jax.lax.dot_general(
        a, b, (((1,), (0,)), ((), ())), preferred_element_type=_F32)


def _dot_bf(a, b):
    return _dot(a.astype(_BF16), b.astype(_BF16))


def _gnn_kernel(g_ref, e_ref, h_ref, Ht_ref, Hb_ref,
                W0_ref, W1_ref, W2_ref, W3_ref,
                nW0_ref, nb0_ref, nW1_ref, nb1_ref,
                nW2_ref, nb2_ref, nW3_ref, nb3_ref,
                out_ref, gbd_ref):
    g = g_ref[...]
    deg = jnp.sum(g, axis=1, keepdims=True)
    deg = jnp.minimum(deg, float(NDEG - 1))

    tq = jax.lax.broadcasted_iota(jnp.int32, (N, CH), 1)
    tw = jax.lax.broadcasted_iota(jnp.int32, (N, CH), 0)
    T = (tq % N == tw).astype(_BF16)
    ri = jax.lax.broadcasted_iota(jnp.int32, (CH, CH), 0)
    ci = jax.lax.broadcasted_iota(jnp.int32, (CH, CH), 1)
    blk = (ri // N == ci // N).astype(_BF16)
    for c in range(P // CH):
        rows = _dot(g_ref[c * CH:(c + 1) * CH, :].astype(_BF16), T)
        gbd_ref[c * CH:(c + 1) * CH, :] = rows.astype(_BF16) * blk

    rl = jax.lax.broadcasted_iota(jnp.int32, (N, EW), 1)
    rw = jax.lax.broadcasted_iota(jnp.int32, (N, EW), 0)
    R = (rl // D_E == rw).astype(_BF16)
    sl = jax.lax.broadcasted_iota(jnp.int32, (EW, D_E), 0)
    sj = jax.lax.broadcasted_iota(jnp.int32, (EW, D_E), 1)
    S = (sl % D_E == sj).astype(_BF16)
    g_rep = _dot(g.astype(_BF16), R)
    prod = g_rep.astype(_BF16) * e_ref[...].astype(_BF16)
    m_e = _dot(prod, S)
    m_e_bf = m_e.astype(_BF16)

    dmask = [(deg == float(d)).astype(_BF16) for d in range(NDEG)]

    el = jax.lax.broadcasted_iota(jnp.int32, (D_E, 8 * D_E), 1)
    ei = jax.lax.broadcasted_iota(jnp.int32, (D_E, 8 * D_E), 0)
    Erep = (el % D_E == ei).astype(_BF16)
    qpat = (jax.lax.broadcasted_iota(jnp.int32, (1, 8 * D_E), 1)
            // D_E).astype(_F32)

    def readout(h_l, W_ref):
        mask = (jnp.sum(h_l, axis=1, keepdims=True) != 0).astype(_F32)
        hm = h_l * mask
        hsum = jnp.sum(hm.reshape(B, N, OUT), axis=1)
        return _dot(hsum, W_ref[...])

    h = h_ref[...]
    aux = readout(h, W0_ref)

    for H_ref, W_ref in ((0, W1_ref), (1, W2_ref), (2, W3_ref)):
        h_bf = h.astype(_BF16)
        mh_cs = [_dot(gbd_ref[c * CH:(c + 1) * CH, :],
                      h_bf[c * CH:(c + 1) * CH, :]) for c in range(P // CH)]
        m_h = jnp.concatenate(mh_cs, axis=0)
        m_h_bf = m_h.astype(_BF16)
        lhs_h = jnp.concatenate([m_h_bf * dm for dm in dmask], axis=1)
        m_e8 = _dot(m_e_bf, Erep).astype(_BF16)
        e_parts = [(m_e8 * (deg - float(8 * q) == qpat).astype(_BF16))
                   for q in range(4)]
        e_parts.append(m_e_bf * dmask[32])
        lhs_e = jnp.concatenate(e_parts, axis=1)
        acc = (_dot(lhs_h, Ht_ref[H_ref]) + _dot(lhs_e, Hb_ref[H_ref]))
        h = jax.nn.sigmoid(acc)
        aux = aux + readout(h, W_ref)

    s = jax.nn.softmax(aux, axis=1)
    x = jax.nn.relu(_dot(s, nW0_ref[...]) + nb0_ref[...])
    x = jax.nn.relu(_dot(x, nW1_ref[...]) + nb1_ref[...])
    x = jax.nn.relu(_dot(x, nW2_ref[...]) + nb2_ref[...])
    x = jax.nn.sigmoid(_dot(x, nW3_ref[...]) + nb3_ref[...])
    out_ref[...] = jax.nn.softmax(x, axis=1)


@functools.partial(jax.jit, static_argnames=("interpret",))
def _run(g, h_in, e, H0, H1, H2, W0, W1, W2, W3,
         nW0, nb0, nW1, nb1, nW2, nb2, nW3, nb3, interpret=False):
    g2 = g.reshape(P, N)
    e2 = e.reshape(P, EW)
    h2 = h_in.reshape(P, D_IN)
    Hs = jnp.stack([H0, H1, H2])
    Ht = Hs[:, :, :D_IN, :].reshape(3, NDEG * D_IN, OUT).astype(_BF16)
    Hb = Hs[:, :, D_IN:, :].reshape(3, NDEG * D_E, OUT).astype(_BF16)
    return pl.pallas_call(
        _gnn_kernel,
        out_shape=jax.ShapeDtypeStruct((B, TGT), _F32),
        scratch_shapes=[pltpu.VMEM((P, CH), _BF16)],
        interpret=interpret,
    )(g2, e2, h2, Ht, Hb, W0, W1, W2, W3,
      nW0, nb0.reshape(1, -1), nW1, nb1.reshape(1, -1),
      nW2, nb2.reshape(1, -1), nW3, nb3.reshape(1, -1))


def kernel(g, h_in, e, H0, H1, H2, W0, W1, W2, W3,
           nW0, nb0, nW1, nb1, nW2, nb2, nW3, nb3):
    return _run(g, h_in, e, H0, H1, H2, W0, W1, W2, W3,
                nW0, nb0, nW1, nb1, nW2, nb2, nW3, nb3)


# speedup vs baseline: 1.7104x; 1.3483x over previous
import functools

import jax
import jax.numpy as jnp
from jax.experimental import pallas as pl
from jax.experimental.pallas import tpu as pltpu

B, N, D_IN, D_E, OUT, TGT = 64, 32, 128, 16, 128, 12
NDEG = 33
P = B * N
MSG = D_IN + D_E
EW = N * D_E
CH = 256
C_LO, C_HI = 8, 24

_F32 = jnp.float32
_BF16 = jnp.bfloat16


def _dot(a, b):
    return jax.lax.dot_general(
        a, b, (((1,), (0,)), ((), ())), preferred_element_type=_F32)


def _gnn_kernel(g_ref, e_ref, h_ref, H0_ref, H1_ref, H2_ref,
                W0_ref, W1_ref, W2_ref, W3_ref,
                nW0_ref, nb0_ref, nW1_ref, nb1_ref,
                nW2_ref, nb2_ref, nW3_ref, nb3_ref,
                out_ref, gbd_ref, acc_ref):
    g = g_ref[...]
    deg = jnp.sum(g, axis=1, keepdims=True)
    deg = jnp.minimum(deg, float(NDEG - 1))
    deg_min = jnp.min(deg)
    deg_max = jnp.max(deg)

    tq = jax.lax.broadcasted_iota(jnp.int32, (N, CH), 1)
    tw = jax.lax.broadcasted_iota(jnp.int32, (N, CH), 0)
    T = (tq % N == tw).astype(_BF16)
    ri = jax.lax.broadcasted_iota(jnp.int32, (CH, CH), 0)
    ci = jax.lax.broadcasted_iota(jnp.int32, (CH, CH), 1)
    blk = (ri // N == ci // N).astype(_BF16)
    for c in range(P // CH):
        rows = _dot(g_ref[c * CH:(c + 1) * CH, :].astype(_BF16), T)
        gbd_ref[c * CH:(c + 1) * CH, :] = rows.astype(_BF16) * blk

    rl = jax.lax.broadcasted_iota(jnp.int32, (N, EW), 1)
    rw = jax.lax.broadcasted_iota(jnp.int32, (N, EW), 0)
    R = (rl // D_E == rw).astype(_BF16)
    sl = jax.lax.broadcasted_iota(jnp.int32, (EW, D_E), 0)
    sj = jax.lax.broadcasted_iota(jnp.int32, (EW, D_E), 1)
    S = (sl % D_E == sj).astype(_BF16)
    g_rep = _dot(g.astype(_BF16), R)
    prod = g_rep.astype(_BF16) * e_ref[...].astype(_BF16)
    m_e = _dot(prod, S)
    m_e_bf = m_e.astype(_BF16)

    dmask = [(deg == float(d)).astype(_BF16) for d in range(NDEG)]

    el = jax.lax.broadcasted_iota(jnp.int32, (D_E, 8 * D_E), 1)
    ei = jax.lax.broadcasted_iota(jnp.int32, (D_E, 8 * D_E), 0)
    Erep = (el % D_E == ei).astype(_BF16)
    qpat = (jax.lax.broadcasted_iota(jnp.int32, (1, 8 * D_E), 1)
            // D_E).astype(_F32)

    def readout(h_l, W_ref):
        mask = (jnp.sum(h_l, axis=1, keepdims=True) != 0).astype(_F32)
        hm = h_l * mask
        hsum = jnp.sum(hm.reshape(B, N, OUT), axis=1)
        return _dot(hsum, W_ref[...])

    h = h_ref[...]
    aux = readout(h, W0_ref)

    for H_ref, W_ref in ((H0_ref, W1_ref), (H1_ref, W2_ref), (H2_ref, W3_ref)):
        h_bf = h.astype(_BF16)
        mh_cs = [_dot(gbd_ref[c * CH:(c + 1) * CH, :],
                      h_bf[c * CH:(c + 1) * CH, :]) for c in range(P // CH)]
        m_h = jnp.concatenate(mh_cs, axis=0)
        m_h_bf = m_h.astype(_BF16)
        m_e8 = _dot(m_e_bf, Erep).astype(_BF16)

        def epiece(q):
            return m_e8 * (deg - float(8 * q) == qpat).astype(_BF16)

        def hpieces(lo, hi):
            return [m_h_bf * dmask[d] for d in range(lo, hi)]

        def hbank(ref, lo, hi):
            return jnp.concatenate(
                [ref[d, :D_IN, :] for d in range(lo, hi)],
                axis=0).astype(_BF16)

        def ebank(ref, lo, hi):
            return jnp.concatenate(
                [ref[d, D_IN:, :] for d in range(lo, hi)],
                axis=0).astype(_BF16)

        lhs_c = jnp.concatenate(
            hpieces(C_LO, C_HI) + [epiece(1), epiece(2)], axis=1)
        rhs_c = jnp.concatenate(
            [hbank(H_ref, C_LO, C_HI), ebank(H_ref, C_LO, C_HI)], axis=0)
        acc_ref[...] = _dot(lhs_c, rhs_c)

        @pl.when(deg_min < float(C_LO))
        def _():
            lhs = jnp.concatenate(hpieces(0, C_LO) + [epiece(0)], axis=1)
            rhs = jnp.concatenate(
                [hbank(H_ref, 0, C_LO), ebank(H_ref, 0, C_LO)], axis=0)
            acc_ref[...] += _dot(lhs, rhs)

        @pl.when(deg_max >= float(C_HI))
        def _():
            lhs = jnp.concatenate(hpieces(C_HI, 32) + [epiece(3)], axis=1)
            rhs = jnp.concatenate(
                [hbank(H_ref, C_HI, 32), ebank(H_ref, C_HI, 32)], axis=0)
            acc_ref[...] += _dot(lhs, rhs)

        @pl.when(deg_max == float(32))
        def _():
            lhs = jnp.concatenate(
                [m_h_bf * dmask[32], m_e_bf * dmask[32]], axis=1)
            acc_ref[...] += _dot(lhs, H_ref[32].astype(_BF16))

        h = jax.nn.sigmoid(acc_ref[...])
        aux = aux + readout(h, W_ref)

    s = jax.nn.softmax(aux, axis=1)
    x = jax.nn.relu(_dot(s, nW0_ref[...]) + nb0_ref[...])
    x = jax.nn.relu(_dot(x, nW1_ref[...]) + nb1_ref[...])
    x = jax.nn.relu(_dot(x, nW2_ref[...]) + nb2_ref[...])
    x = jax.nn.sigmoid(_dot(x, nW3_ref[...]) + nb3_ref[...])
    out_ref[...] = jax.nn.softmax(x, axis=1)


@functools.partial(jax.jit, static_argnames=("interpret",))
def _run(g, h_in, e, H0, H1, H2, W0, W1, W2, W3,
         nW0, nb0, nW1, nb1, nW2, nb2, nW3, nb3, interpret=False):
    g2 = g.reshape(P, N)
    e2 = e.reshape(P, EW)
    h2 = h_in.reshape(P, D_IN)
    return pl.pallas_call(
        _gnn_kernel,
        out_shape=jax.ShapeDtypeStruct((B, TGT), _F32),
        scratch_shapes=[pltpu.VMEM((P, CH), _BF16),
                        pltpu.VMEM((P, OUT), _F32)],
        interpret=interpret,
    )(g2, e2, h2, H0, H1, H2, W0, W1, W2, W3,
      nW0, nb0.reshape(1, -1), nW1, nb1.reshape(1, -1),
      nW2, nb2.reshape(1, -1), nW3, nb3.reshape(1, -1))


def kernel(g, h_in, e, H0, H1, H2, W0, W1, W2, W3,
           nW0, nb0, nW1, nb1, nW2, nb2, nW3, nb3):
    return _run(g, h_in, e, H0, H1, H2, W0, W1, W2, W3,
                nW0, nb0, nW1, nb1, nW2, nb2, nW3, nb3)
